# Initial kernel scaffold; baseline (speedup 1.0000x reference)
import jax
import jax.numpy as jnp
from jax.experimental import pallas as pl


def kernel(x, W, b):
    raise NotImplementedError("write your pallas kernel here")


# speedup vs baseline: 1.1367x; 1.1367x over previous
import functools

import jax
import jax.numpy as jnp
from jax import lax
from jax.experimental import pallas as pl
from jax.experimental.pallas import tpu as pltpu

INPUT_SIZE = 2048
NUM_EXPERTS = 64
TOP_K = 8
TOKEN_TILE = 512


def _gate_kernel(x_ref, wt_ref, b_ref, out_ref):
    logits = jnp.dot(x_ref[...], wt_ref[...], preferred_element_type=jnp.float32)
    logits = logits + b_ref[...]

    T = logits.shape[0]
    idx = lax.broadcasted_iota(jnp.int32, (T, NUM_EXPERTS), 1)
    neg_inf = jnp.float32(-jnp.inf)

    work = logits
    num = jnp.zeros_like(logits)
    den = jnp.zeros((T, 1), jnp.float32)
    m0 = None
    for _ in range(TOP_K):
        m = jnp.max(work, axis=-1, keepdims=True)
        first = jnp.min(
            jnp.where(work == m, idx, NUM_EXPERTS), axis=-1, keepdims=True
        )
        pick = idx == first
        if m0 is None:
            m0 = m
        e = jnp.exp(m - m0)
        num = jnp.where(pick, e, num)
        den = den + e
        work = jnp.where(pick, neg_inf, work)
    out_ref[...] = num / den


@jax.jit
def kernel(x, W, b):
    B, S, D = x.shape
    tokens = B * S
    x2 = x.reshape(tokens, D)
    wt = W.T
    b2 = b.reshape(1, NUM_EXPERTS)

    grid = (tokens // TOKEN_TILE,)
    out = pl.pallas_call(
        _gate_kernel,
        grid=grid,
        in_specs=[
            pl.BlockSpec((TOKEN_TILE, D), lambda i: (i, 0)),
            pl.BlockSpec((D, NUM_EXPERTS), lambda i: (0, 0)),
            pl.BlockSpec((1, NUM_EXPERTS), lambda i: (0, 0)),
        ],
        out_specs=pl.BlockSpec((TOKEN_TILE, NUM_EXPERTS), lambda i: (i, 0)),
        out_shape=jax.ShapeDtypeStruct((tokens, NUM_EXPERTS), jnp.float32),
    )(x2, wt, b2)
    return out.reshape(B, S, NUM_EXPERTS)


# speedup vs baseline: 1.6161x; 1.4217x over previous
import functools

import jax
import jax.numpy as jnp
from jax import lax
from jax.experimental import pallas as pl
from jax.experimental.pallas import tpu as pltpu

INPUT_SIZE = 2048
NUM_EXPERTS = 64
TOP_K = 8
TOKEN_TILE = 512


def _gate_kernel(x_ref, wt_ref, b_ref, out_ref):
    logits = jnp.dot(x_ref[...], wt_ref[...], preferred_element_type=jnp.float32)
    logits = logits + b_ref[...]

    neg_inf = jnp.float32(-jnp.inf)

    work = logits
    m0 = None
    for _ in range(TOP_K):
        m = jnp.max(work, axis=-1, keepdims=True)
        if m0 is None:
            m0 = m
        work = jnp.where(work == m, neg_inf, work)
    q = jnp.where(logits >= m, logits, neg_inf)
    p = jnp.exp(q - m0)
    den = jnp.sum(p, axis=-1, keepdims=True)
    out_ref[...] = p / den


@jax.jit
def kernel(x, W, b):
    B, S, D = x.shape
    tokens = B * S
    x2 = x.reshape(tokens, D)
    wt = W.T
    b2 = b.reshape(1, NUM_EXPERTS)

    grid = (tokens // TOKEN_TILE,)
    out = pl.pallas_call(
        _gate_kernel,
        grid=grid,
        in_specs=[
            pl.BlockSpec((TOKEN_TILE, D), lambda i: (i, 0)),
            pl.BlockSpec((D, NUM_EXPERTS), lambda i: (0, 0)),
            pl.BlockSpec((1, NUM_EXPERTS), lambda i: (0, 0)),
        ],
        out_specs=pl.BlockSpec((TOKEN_TILE, NUM_EXPERTS), lambda i: (i, 0)),
        out_shape=jax.ShapeDtypeStruct((tokens, NUM_EXPERTS), jnp.float32),
    )(x2, wt, b2)
    return out.reshape(B, S, NUM_EXPERTS)
